# baseline (speedup 1.0000x reference)
import functools

import jax
import jax.numpy as jnp
from jax import lax
from jax.experimental import pallas as pl
from jax.experimental.pallas import tpu as pltpu
from jax.experimental.pallas import tpu_sc as plsc

B = 16384
EMB = 32
H1, H2 = 256, 128
FEAT = 163
NC, NS, LANES = 2, 16, 16
NW = NC * NS
SPW = B // NW
TITLE_K = 20
GENRE_K = 4
GENRE_V, LANG_V, YEAR_V, RUNTIME_V = 21, 24, 13, 32
TITLE_CHUNK = 64
N_TCHUNK = SPW // TITLE_CHUNK
ROWS = TITLE_CHUNK * TITLE_K


def _tree_sum(vs):
  while len(vs) > 1:
    nxt = [vs[i] + vs[i + 1] for i in range(0, len(vs) - 1, 2)]
    if len(vs) % 2:
      nxt.append(vs[-1])
    vs = nxt
  return vs[0]


def _make_title_kernel():
  mesh = plsc.VectorSubcoreMesh(core_axis_name="c", subcore_axis_name="s",
                                num_cores=NC, num_subcores=NS)

  @functools.partial(
      pl.kernel,
      out_type=jax.ShapeDtypeStruct((B, EMB), jnp.float32),
      mesh=mesh,
      scratch_types=[
          pltpu.VMEM((SPW * TITLE_K,), jnp.int32),
          pltpu.VMEM((ROWS, EMB), jnp.float32),
          pltpu.VMEM((ROWS, EMB), jnp.float32),
          pltpu.VMEM((SPW, EMB), jnp.float32),
          pltpu.SemaphoreType.DMA,
          pltpu.SemaphoreType.DMA,
      ],
      compiler_params=pltpu.CompilerParams(use_tc_tiling_on_sc=False),
  )
  def pool(title_idx_h, title_h, out_h, tidx_v, buf0, buf1, pool_v,
           sem0, sem1):
    wid = lax.axis_index("s") * NC + lax.axis_index("c")
    base = wid * SPW
    bufs = (buf0, buf1)
    sems = (sem0, sem1)

    pltpu.sync_copy(title_idx_h.at[pl.ds(base * TITLE_K, SPW * TITLE_K)],
                    tidx_v)

    def start(c):
      return pltpu.async_copy(
          title_h.at[tidx_v.at[pl.ds(c * ROWS, ROWS)]],
          bufs[c % 2], sems[c % 2])

    cp = start(0)
    for c in range(N_TCHUNK):
      nxt = start(c + 1) if c + 1 < N_TCHUNK else None
      cp.wait()
      rows_v = bufs[c % 2]

      def tbody(j, _, c=c, rows_v=rows_v):
        o = c * TITLE_CHUNK + j
        for h in range(EMB // LANES):
          sl = pl.ds(h * LANES, LANES)
          vs = [rows_v[j * TITLE_K + t, sl] for t in range(TITLE_K)]
          pool_v[o, sl] = _tree_sum(vs)
        return 0

      lax.fori_loop(0, TITLE_CHUNK, tbody, 0)
      cp = nxt
    pltpu.sync_copy(pool_v, out_h.at[pl.ds(base, SPW)])

  return pool


_MLP_BLK = 2048


def _mlp_body(tp_ref, aux_ref, gtab_ref, ltab_ref, ytab_ref, rtab_ref,
              w1_ref, b1_ref, w2_ref, b2_ref, w3_ref, o_ref):
  f32 = jnp.float32
  i32 = jnp.int32
  aux = aux_ref[...]
  iog = lax.broadcasted_iota(i32, (_MLP_BLK, GENRE_V), 1)
  cnt = _tree_sum([(aux[t].astype(i32)[:, None] == iog).astype(f32)
                   for t in range(4)])
  ge = jnp.dot(cnt, gtab_ref[...], preferred_element_type=f32)

  embs = [tp_ref[...] * (1.0 / TITLE_K), ge * (1.0 / GENRE_K)]
  for t, vocab, tab in ((4, LANG_V, ltab_ref), (5, YEAR_V, ytab_ref),
                        (6, RUNTIME_V, rtab_ref)):
    io = lax.broadcasted_iota(i32, (_MLP_BLK, vocab), 1)
    oh = (aux[t].astype(i32)[:, None] == io).astype(f32)
    embs.append(jnp.dot(oh, tab[...], preferred_element_type=f32))

  feat = jnp.concatenate(embs, axis=1)
  acc = jnp.dot(feat, w1_ref[0:5 * EMB], preferred_element_type=f32)
  acc = acc + aux[7][:, None] * w1_ref[5 * EMB + 0:5 * EMB + 1]
  acc = acc + aux[8][:, None] * w1_ref[5 * EMB + 1:5 * EMB + 2]
  acc = acc + aux[9][:, None] * w1_ref[5 * EMB + 2:5 * EMB + 3]
  h = jnp.maximum(acc + b1_ref[...][None, :], 0.0)
  h = jnp.maximum(jnp.dot(h, w2_ref[...], preferred_element_type=f32)
                  + b2_ref[...][None, :], 0.0)
  o_ref[...] = jnp.dot(h, w3_ref[...], preferred_element_type=f32)


def _mlp(title_pool, aux, genre_tab, lang_tab, year_tab, runtime_tab,
         W1, b1, W2, b2, W3):
  nblk = B // _MLP_BLK
  full2 = lambda a, b: pl.BlockSpec((a, b), lambda i: (0, 0))
  return pl.pallas_call(
      _mlp_body,
      grid=(nblk,),
      in_specs=[
          pl.BlockSpec((_MLP_BLK, EMB), lambda i: (i, 0)),
          pl.BlockSpec((10, _MLP_BLK), lambda i: (0, i)),
          full2(GENRE_V, EMB), full2(LANG_V, EMB), full2(YEAR_V, EMB),
          full2(RUNTIME_V, EMB),
          full2(FEAT, H1),
          pl.BlockSpec((H1,), lambda i: (0,)),
          full2(H1, H2),
          pl.BlockSpec((H2,), lambda i: (0,)),
          full2(H2, EMB),
      ],
      out_specs=pl.BlockSpec((_MLP_BLK, EMB), lambda i: (i, 0)),
      out_shape=jax.ShapeDtypeStruct((B, EMB), jnp.float32),
  )(title_pool, aux, genre_tab, lang_tab, year_tab, runtime_tab,
    W1, b1, W2, b2, W3)


def kernel(movie_title_vec, genres_encoded, language, year_released, runtime,
           popularity, vote_count, vote_average,
           title_tab, genre_tab, lang_tab, year_tab, runtime_tab,
           W1, b1, W2, b2, W3):
  f32 = jnp.float32
  i32 = jnp.int32
  title_idx = movie_title_vec.reshape(-1).astype(i32)

  title_pool = _make_title_kernel()(title_idx, title_tab)

  aux = jnp.concatenate(
      [genres_encoded.T.astype(f32), language.T.astype(f32),
       year_released.T.astype(f32), runtime.T.astype(f32),
       popularity.T, vote_count.T, vote_average.T], axis=0)

  return _mlp(title_pool, aux, genre_tab, lang_tab, year_tab, runtime_tab,
              W1, b1, W2, b2, W3)


# speedup vs baseline: 1.0020x; 1.0020x over previous
import functools

import jax
import jax.numpy as jnp
from jax import lax
from jax.experimental import pallas as pl
from jax.experimental.pallas import tpu as pltpu
from jax.experimental.pallas import tpu_sc as plsc

B = 16384
EMB = 32
H1, H2 = 256, 128
FEAT = 163
NC, NS, LANES = 2, 16, 16
NW = NC * NS
SPW = B // NW
TITLE_K = 20
GENRE_K = 4
GENRE_V, LANG_V, YEAR_V, RUNTIME_V = 21, 24, 13, 32
TITLE_CHUNK = 64
N_TCHUNK = SPW // TITLE_CHUNK
ROWS = TITLE_CHUNK * TITLE_K


def _tree_sum(vs):
  while len(vs) > 1:
    nxt = [vs[i] + vs[i + 1] for i in range(0, len(vs) - 1, 2)]
    if len(vs) % 2:
      nxt.append(vs[-1])
    vs = nxt
  return vs[0]


def _make_title_kernel():
  mesh = plsc.VectorSubcoreMesh(core_axis_name="c", subcore_axis_name="s",
                                num_cores=NC, num_subcores=NS)

  @functools.partial(
      pl.kernel,
      out_type=jax.ShapeDtypeStruct((B, EMB), jnp.float32),
      mesh=mesh,
      scratch_types=[
          pltpu.VMEM((SPW * TITLE_K,), jnp.int32),
          pltpu.VMEM((ROWS, EMB), jnp.float32),
          pltpu.VMEM((ROWS, EMB), jnp.float32),
          pltpu.VMEM((SPW, EMB), jnp.float32),
          pltpu.SemaphoreType.DMA,
          pltpu.SemaphoreType.DMA,
      ],
      compiler_params=pltpu.CompilerParams(use_tc_tiling_on_sc=False),
  )
  def pool(title_idx_h, title_h, out_h, tidx_v, buf0, buf1, pool_v,
           sem0, sem1):
    wid = lax.axis_index("s") * NC + lax.axis_index("c")
    base = wid * SPW
    bufs = (buf0, buf1)
    sems = (sem0, sem1)

    pltpu.sync_copy(title_idx_h.at[pl.ds(base * TITLE_K, SPW * TITLE_K)],
                    tidx_v)

    def start(c):
      return pltpu.async_copy(
          title_h.at[tidx_v.at[pl.ds(c * ROWS, ROWS)]],
          bufs[c % 2], sems[c % 2])

    cp = start(0)
    for c in range(N_TCHUNK):
      nxt = start(c + 1) if c + 1 < N_TCHUNK else None
      cp.wait()
      rows_v = bufs[c % 2]

      def tbody(j, _, c=c, rows_v=rows_v):
        o = c * TITLE_CHUNK + j
        for h in range(EMB // LANES):
          sl = pl.ds(h * LANES, LANES)
          vs = [rows_v[j * TITLE_K + t, sl] for t in range(TITLE_K)]
          pool_v[o, sl] = _tree_sum(vs)
        return 0

      lax.fori_loop(0, TITLE_CHUNK, tbody, 0)
      cp = nxt
    pltpu.sync_copy(pool_v, out_h.at[pl.ds(base, SPW)])

  return pool


_MLP_BLK = 2048


def _mlp_body(tp_ref, aux_ref, gtab_ref, ltab_ref, ytab_ref, rtab_ref,
              w1_ref, b1_ref, w2_ref, b2_ref, w3_ref, o_ref):
  f32 = jnp.float32
  i32 = jnp.int32
  aux = aux_ref[...]
  iog = lax.broadcasted_iota(i32, (_MLP_BLK, GENRE_V), 1)
  cnt = _tree_sum([(aux[t].astype(i32)[:, None] == iog).astype(f32)
                   for t in range(4)])
  ge = jnp.dot(cnt, gtab_ref[...], preferred_element_type=f32)

  embs = [tp_ref[...] * (1.0 / TITLE_K), ge * (1.0 / GENRE_K)]
  for t, vocab, tab in ((4, LANG_V, ltab_ref), (5, YEAR_V, ytab_ref),
                        (6, RUNTIME_V, rtab_ref)):
    io = lax.broadcasted_iota(i32, (_MLP_BLK, vocab), 1)
    oh = (aux[t].astype(i32)[:, None] == io).astype(f32)
    embs.append(jnp.dot(oh, tab[...], preferred_element_type=f32))

  feat = jnp.concatenate(embs, axis=1)
  acc = jnp.dot(feat, w1_ref[0:5 * EMB], preferred_element_type=f32)
  acc = acc + aux[7][:, None] * w1_ref[5 * EMB + 0:5 * EMB + 1]
  acc = acc + aux[8][:, None] * w1_ref[5 * EMB + 1:5 * EMB + 2]
  acc = acc + aux[9][:, None] * w1_ref[5 * EMB + 2:5 * EMB + 3]
  h = jnp.maximum(acc + b1_ref[...][None, :], 0.0)
  h = jnp.maximum(jnp.dot(h, w2_ref[...], preferred_element_type=f32)
                  + b2_ref[...][None, :], 0.0)
  o_ref[...] = jnp.dot(h, w3_ref[...], preferred_element_type=f32)


def _mlp(title_pool, aux, genre_tab, lang_tab, year_tab, runtime_tab,
         W1, b1, W2, b2, W3):
  nblk = B // _MLP_BLK
  full2 = lambda a, b: pl.BlockSpec((a, b), lambda i: (0, 0))
  return pl.pallas_call(
      _mlp_body,
      grid=(nblk,),
      in_specs=[
          pl.BlockSpec((_MLP_BLK, EMB), lambda i: (i, 0)),
          pl.BlockSpec((10, _MLP_BLK), lambda i: (0, i)),
          full2(GENRE_V, EMB), full2(LANG_V, EMB), full2(YEAR_V, EMB),
          full2(RUNTIME_V, EMB),
          full2(FEAT, H1),
          pl.BlockSpec((H1,), lambda i: (0,)),
          full2(H1, H2),
          pl.BlockSpec((H2,), lambda i: (0,)),
          full2(H2, EMB),
      ],
      out_specs=pl.BlockSpec((_MLP_BLK, EMB), lambda i: (i, 0)),
      out_shape=jax.ShapeDtypeStruct((B, EMB), jnp.float32),
  )(title_pool, aux, genre_tab, lang_tab, year_tab, runtime_tab,
    W1, b1, W2, b2, W3)


def kernel(movie_title_vec, genres_encoded, language, year_released, runtime,
           popularity, vote_count, vote_average,
           title_tab, genre_tab, lang_tab, year_tab, runtime_tab,
           W1, b1, W2, b2, W3):
  f32 = jnp.float32
  i32 = jnp.int32
  title_idx = movie_title_vec.reshape(-1).astype(i32)

  aux = jnp.concatenate(
      [genres_encoded.T.astype(f32), language.T.astype(f32),
       year_released.T.astype(f32), runtime.T.astype(f32),
       popularity.T, vote_count.T, vote_average.T], axis=0)

  title_pool = _make_title_kernel()(title_idx, title_tab)

  return _mlp(title_pool, aux, genre_tab, lang_tab, year_tab, runtime_tab,
              W1, b1, W2, b2, W3)
